# baseline (speedup 1.0000x reference)
import functools

import jax
import jax.numpy as jnp
import numpy as np
from jax.experimental import pallas as pl
from jax.experimental.pallas import tpu as pltpu

_N = 20000
_NB = 20
_BLK = 1024
_LANES = 128
_NPAD = _NB * _BLK
_K = 5000
_OUT = 1000
_NEG_INF = float("-inf")
_MAX_RATIO = float(np.abs(np.log(np.float32(16.0 / 1000.0))))
_SIGMA = 0.5


def _nms_body(score_ref, anch_ref, delt_ref, out_ref,
              s_ref, s2_ref, sb_ref, y1_ref, x1_ref, y2_ref, x2_ref, ar_ref):
    f32 = jnp.float32
    sc = score_ref[:]

    a0 = anch_ref[0]
    a1 = anch_ref[1]
    a2 = anch_ref[2]
    a3 = anch_ref[3]
    dx = delt_ref[0] * f32(0.1)
    dy = delt_ref[1] * f32(0.1)
    dw = jnp.clip(delt_ref[2] * f32(0.2), -_MAX_RATIO, _MAX_RATIO)
    dh = jnp.clip(delt_ref[3] * f32(0.2), -_MAX_RATIO, _MAX_RATIO)
    widths = a2 - a0
    heights = a3 - a1
    ctr_x = a0 + f32(0.5) * widths
    ctr_y = a1 + f32(0.5) * heights
    pred_ctr_x = dx * widths + ctr_x
    pred_ctr_y = dy * heights + ctr_y
    pred_w = widths * jnp.exp(dw)
    pred_h = heights * jnp.exp(dh)
    x1 = jnp.clip(pred_ctr_x - f32(0.5) * pred_w, 0.0, 1.0)
    y1 = jnp.clip(pred_ctr_y - f32(0.5) * pred_h, 0.0, 1.0)
    x2 = jnp.clip(pred_ctr_x + f32(0.5) * pred_w, 0.0, 1.0)
    y2 = jnp.clip(pred_ctr_y + f32(0.5) * pred_h, 0.0, 1.0)
    areas = (y2 - y1) * (x2 - x1)
    y1_ref[:] = y1
    x1_ref[:] = x1
    y2_ref[:] = y2
    x2_ref[:] = x2
    ar_ref[:] = areas

    jb = jax.lax.broadcasted_iota(jnp.int32, (_NB, 8, _LANES), 0)
    sub = jax.lax.broadcasted_iota(jnp.int32, (_NB, 8, _LANES), 1)
    lane = jax.lax.broadcasted_iota(jnp.int32, (_NB, 8, _LANES), 2)
    idx3 = jb * _BLK + sub * _LANES + lane
    valid = idx3 < _N
    sbits = jnp.where(valid, jax.lax.bitcast_convert_type(sc, jnp.int32),
                      jnp.int32(-1))
    sb_ref[:] = sbits

    def _bs_val(_, c):
        lo, hi = c
        mid = (lo + hi) // 2
        cnt = jnp.sum((sbits >= mid).astype(jnp.int32))
        big = cnt >= _K
        return jnp.where(big, mid, lo), jnp.where(big, hi, mid)

    v, _ = jax.lax.fori_loop(
        0, 31, _bs_val, (jnp.int32(0), jnp.int32(0x3F800000)))
    cnt_gt = jnp.sum((sbits > v).astype(jnp.int32))
    need = _K - cnt_gt
    eq = sbits == v

    def _bs_idx(_, c):
        lo, hi = c
        mid = (lo + hi) // 2
        g = jnp.sum((eq & (idx3 < mid)).astype(jnp.int32))
        enough = g >= need
        return jnp.where(enough, lo, mid), jnp.where(enough, mid, hi)

    _, cut = jax.lax.fori_loop(
        0, 16, _bs_idx, (jnp.int32(0), jnp.int32(_NPAD)))
    sel = (sbits > v) | (eq & (idx3 < cut))
    s0 = jnp.where(sel, sc, f32(_NEG_INF))
    s_ref[:] = s0

    pos = jax.lax.broadcasted_iota(jnp.int32, (8, _LANES), 0) * _LANES + \
        jax.lax.broadcasted_iota(jnp.int32, (8, _LANES), 1)
    M0 = jnp.full((8, _LANES), f32(_NEG_INF))
    SB0 = jnp.full((8, _LANES), jnp.int32(-1))
    PS0 = jnp.zeros((8, _LANES), jnp.int32)
    BY10 = jnp.zeros((8, _LANES), f32)
    BX10 = jnp.zeros((8, _LANES), f32)
    BY20 = jnp.zeros((8, _LANES), f32)
    BX20 = jnp.zeros((8, _LANES), f32)
    for j in range(_NB):
        vj = s0[j]
        bj = sbits[j]
        better = (vj > M0) | ((vj == M0) & (bj > SB0))
        M0 = jnp.where(better, vj, M0)
        SB0 = jnp.where(better, bj, SB0)
        PS0 = jnp.where(better, jnp.int32(j) * _BLK + pos, PS0)
        BY10 = jnp.where(better, y1[j], BY10)
        BX10 = jnp.where(better, x1[j], BX10)
        BY20 = jnp.where(better, y2[j], BY20)
        BX20 = jnp.where(better, x2[j], BX20)
    m0 = jnp.max(M0)

    zeros8 = jnp.zeros((8, _LANES), jnp.float32)
    _NACC = 4

    ones128 = jnp.ones((_LANES, _LANES), f32)
    ones8 = jnp.ones((8, 8), f32)
    _dn_r = (((1,), (0,)), ((), ()))

    def _bcsum(a):
        rs = jax.lax.dot_general(a, ones128, _dn_r,
                                 preferred_element_type=f32)
        return jax.lax.dot_general(ones8, rs, _dn_r,
                                   preferred_element_type=f32)

    def _one(i, carry, src_ref, dst_ref):
        ox1, oy1, ox2, oy2, M, SB, PS, BY1, BX1, BY2, BX2, m = carry
        tiedp = M == m
        tcnt = jnp.sum(tiedp.astype(jnp.int32))
        pidx0 = jnp.min(jnp.where(tiedp, PS, jnp.int32(_NPAD)))
        tf = tiedp.astype(f32)
        sy1 = _bcsum(BY1 * tf)
        sx1 = _bcsum(BX1 * tf)
        sy2 = _bcsum(BY2 * tf)
        sx2 = _bcsum(BX2 * tf)

        def _exact(_):
            msb = jnp.max(jnp.where(tiedp, SB, jnp.int32(-1)))
            tied2 = tiedp & (SB == msb)
            p = jnp.min(jnp.where(tied2, PS, jnp.int32(_NPAD)))
            w = (PS == p) & tied2
            wf = w.astype(f32)
            full = lambda s: jnp.full((8, _LANES), s, f32)
            return (p, full(jnp.sum(BY1 * wf)), full(jnp.sum(BX1 * wf)),
                    full(jnp.sum(BY2 * wf)), full(jnp.sum(BX2 * wf)))

        def _fast(_):
            return (pidx0, sy1, sx1, sy2, sx2)

        pidx, by1, bx1, by2, bx2 = jax.lax.cond(tcnt == 1, _fast, _exact, 0)
        ok = m > f32(_NEG_INF)
        here = pos == i
        zf = f32(0.0)
        ox1 = jnp.where(here, jnp.where(ok, bx1, zf), ox1)
        oy1 = jnp.where(here, jnp.where(ok, by1, zf), oy1)
        ox2 = jnp.where(here, jnp.where(ok, bx2, zf), ox2)
        oy2 = jnp.where(here, jnp.where(ok, by2, zf), oy2)
        area_a = (by2 - by1) * (bx2 - bx1)
        Ms = [jnp.full((8, _LANES), f32(_NEG_INF)) for _ in range(_NACC)]
        SBs = [jnp.full((8, _LANES), jnp.int32(-1)) for _ in range(_NACC)]
        PSs = [jnp.zeros((8, _LANES), jnp.int32) for _ in range(_NACC)]
        Y1s = [jnp.zeros((8, _LANES), f32) for _ in range(_NACC)]
        X1s = [jnp.zeros((8, _LANES), f32) for _ in range(_NACC)]
        Y2s = [jnp.zeros((8, _LANES), f32) for _ in range(_NACC)]
        X2s = [jnp.zeros((8, _LANES), f32) for _ in range(_NACC)]
        for j in range(_NB):
            a = j % _NACC
            sj = src_ref[j]
            cy1 = y1_ref[j]
            cx1 = x1_ref[j]
            cy2 = y2_ref[j]
            cx2 = x2_ref[j]
            yy1 = jnp.maximum(by1, cy1)
            xx1 = jnp.maximum(bx1, cx1)
            yy2 = jnp.minimum(by2, cy2)
            xx2 = jnp.minimum(bx2, cx2)
            inter = jnp.maximum(yy2 - yy1, f32(0.0)) * \
                jnp.maximum(xx2 - xx1, f32(0.0))
            ious = inter / (area_a + ar_ref[j] - inter + f32(1e-8))
            decay = jnp.exp(-(ious * ious) / f32(_SIGMA))
            v2 = sj * decay
            v2 = jnp.where(pos == pidx - j * _BLK, f32(_NEG_INF), v2)
            dst_ref[j] = v2
            bj = sb_ref[j]
            better = (v2 > Ms[a]) | ((v2 == Ms[a]) & (bj > SBs[a]))
            Ms[a] = jnp.where(better, v2, Ms[a])
            SBs[a] = jnp.where(better, bj, SBs[a])
            PSs[a] = jnp.where(better, jnp.int32(j) * _BLK + pos, PSs[a])
            Y1s[a] = jnp.where(better, cy1, Y1s[a])
            X1s[a] = jnp.where(better, cx1, X1s[a])
            Y2s[a] = jnp.where(better, cy2, Y2s[a])
            X2s[a] = jnp.where(better, cx2, X2s[a])
        Mn, SBn, PSn = Ms[0], SBs[0], PSs[0]
        Y1n, X1n, Y2n, X2n = Y1s[0], X1s[0], Y2s[0], X2s[0]
        for a in range(1, _NACC):
            better = (Ms[a] > Mn) | ((Ms[a] == Mn) & (
                (SBs[a] > SBn) | ((SBs[a] == SBn) & (PSs[a] < PSn))))
            Mn = jnp.where(better, Ms[a], Mn)
            SBn = jnp.where(better, SBs[a], SBn)
            PSn = jnp.where(better, PSs[a], PSn)
            Y1n = jnp.where(better, Y1s[a], Y1n)
            X1n = jnp.where(better, X1s[a], X1n)
            Y2n = jnp.where(better, Y2s[a], Y2n)
            X2n = jnp.where(better, X2s[a], X2n)
        mn = jnp.max(jnp.maximum(jnp.maximum(Ms[0], Ms[1]),
                                 jnp.maximum(Ms[2], Ms[3])))
        return ox1, oy1, ox2, oy2, Mn, SBn, PSn, Y1n, X1n, Y2n, X2n, mn

    def _iter2(k, carry):
        carry = _one(2 * k, carry, s_ref, s2_ref)
        return _one(2 * k + 1, carry, s2_ref, s_ref)

    ox1, oy1, ox2, oy2 = jax.lax.fori_loop(
        0, _OUT // 2, _iter2,
        (zeros8, zeros8, zeros8, zeros8, M0, SB0, PS0,
         BY10, BX10, BY20, BX20, m0))[:4]
    out_ref[0] = ox1
    out_ref[1] = oy1
    out_ref[2] = ox2
    out_ref[3] = oy2


@functools.partial(jax.jit, static_argnames=("interpret",))
def _run(rpn_score, rpn_regress, anchors, interpret=False):
    score = rpn_score.reshape(_N)
    pad = _NPAD - _N
    score_p = jnp.pad(score, (0, pad)).reshape(_NB, 8, _LANES)
    anch_p = jnp.pad(anchors, ((0, pad), (0, 0))).T.reshape(4, _NB, 8, _LANES)
    delt_p = jnp.pad(rpn_regress.reshape(_N, 4),
                     ((0, pad), (0, 0))).T.reshape(4, _NB, 8, _LANES)
    out = pl.pallas_call(
        _nms_body,
        out_shape=jax.ShapeDtypeStruct((4, 8, _LANES), jnp.float32),
        scratch_shapes=[
            pltpu.VMEM((_NB, 8, _LANES), jnp.float32),
            pltpu.VMEM((_NB, 8, _LANES), jnp.float32),
            pltpu.VMEM((_NB, 8, _LANES), jnp.int32),
            pltpu.VMEM((_NB, 8, _LANES), jnp.float32),
            pltpu.VMEM((_NB, 8, _LANES), jnp.float32),
            pltpu.VMEM((_NB, 8, _LANES), jnp.float32),
            pltpu.VMEM((_NB, 8, _LANES), jnp.float32),
            pltpu.VMEM((_NB, 8, _LANES), jnp.float32),
        ],
        interpret=interpret,
    )(score_p, anch_p, delt_p)
    out = out.reshape(4, 8 * _LANES)[:, :_OUT]
    return jax.lax.stop_gradient(out.T.reshape(1, _OUT, 4))


def kernel(rpn_score, rpn_regress, anchors):
    return _run(rpn_score, rpn_regress, anchors)


# speedup vs baseline: 1.8382x; 1.8382x over previous
import functools

import jax
import jax.numpy as jnp
import numpy as np
from jax.experimental import pallas as pl
from jax.experimental.pallas import tpu as pltpu

_N = 20000
_NB = 20
_BLK = 1024
_LANES = 128
_NPAD = _NB * _BLK
_K = 5000
_OUT = 1000
_NEG_INF = float("-inf")
_MAX_RATIO = float(np.abs(np.log(np.float32(16.0 / 1000.0))))
_SIGMA = 0.5


def _nms_body(score_ref, anch_ref, delt_ref, out_ref,
              s_ref, s2_ref, sb_ref, y1_ref, x1_ref, y2_ref, x2_ref, ar_ref):
    f32 = jnp.float32
    sc = score_ref[:]

    a0 = anch_ref[0]
    a1 = anch_ref[1]
    a2 = anch_ref[2]
    a3 = anch_ref[3]
    dx = delt_ref[0] * f32(0.1)
    dy = delt_ref[1] * f32(0.1)
    dw = jnp.clip(delt_ref[2] * f32(0.2), -_MAX_RATIO, _MAX_RATIO)
    dh = jnp.clip(delt_ref[3] * f32(0.2), -_MAX_RATIO, _MAX_RATIO)
    widths = a2 - a0
    heights = a3 - a1
    ctr_x = a0 + f32(0.5) * widths
    ctr_y = a1 + f32(0.5) * heights
    pred_ctr_x = dx * widths + ctr_x
    pred_ctr_y = dy * heights + ctr_y
    pred_w = widths * jnp.exp(dw)
    pred_h = heights * jnp.exp(dh)
    x1 = jnp.clip(pred_ctr_x - f32(0.5) * pred_w, 0.0, 1.0)
    y1 = jnp.clip(pred_ctr_y - f32(0.5) * pred_h, 0.0, 1.0)
    x2 = jnp.clip(pred_ctr_x + f32(0.5) * pred_w, 0.0, 1.0)
    y2 = jnp.clip(pred_ctr_y + f32(0.5) * pred_h, 0.0, 1.0)
    areas = (y2 - y1) * (x2 - x1)
    y1_ref[:] = y1
    x1_ref[:] = x1
    y2_ref[:] = y2
    x2_ref[:] = x2
    ar_ref[:] = areas

    jb = jax.lax.broadcasted_iota(jnp.int32, (_NB, 8, _LANES), 0)
    sub = jax.lax.broadcasted_iota(jnp.int32, (_NB, 8, _LANES), 1)
    lane = jax.lax.broadcasted_iota(jnp.int32, (_NB, 8, _LANES), 2)
    idx3 = jb * _BLK + sub * _LANES + lane
    valid = idx3 < _N
    sbits = jnp.where(valid, jax.lax.bitcast_convert_type(sc, jnp.int32),
                      jnp.int32(-1))
    sb_ref[:] = sbits

    def _bs_val(_, c):
        lo, hi = c
        mid = (lo + hi) // 2
        cnt = jnp.sum((sbits >= mid).astype(jnp.int32))
        big = cnt >= _K
        return jnp.where(big, mid, lo), jnp.where(big, hi, mid)

    v, _ = jax.lax.fori_loop(
        0, 31, _bs_val, (jnp.int32(0), jnp.int32(0x3F800000)))
    cnt_gt = jnp.sum((sbits > v).astype(jnp.int32))
    need = _K - cnt_gt
    eq = sbits == v

    def _bs_idx(_, c):
        lo, hi = c
        mid = (lo + hi) // 2
        g = jnp.sum((eq & (idx3 < mid)).astype(jnp.int32))
        enough = g >= need
        return jnp.where(enough, lo, mid), jnp.where(enough, mid, hi)

    _, cut = jax.lax.fori_loop(
        0, 16, _bs_idx, (jnp.int32(0), jnp.int32(_NPAD)))
    sel = (sbits > v) | (eq & (idx3 < cut))
    s0 = jnp.where(sel, sc, f32(_NEG_INF))
    s_ref[:] = s0

    pos = jax.lax.broadcasted_iota(jnp.int32, (8, _LANES), 0) * _LANES + \
        jax.lax.broadcasted_iota(jnp.int32, (8, _LANES), 1)
    M0 = jnp.full((8, _LANES), f32(_NEG_INF))
    SB0 = jnp.full((8, _LANES), jnp.int32(-1))
    PS0 = jnp.zeros((8, _LANES), jnp.int32)
    BY10 = jnp.zeros((8, _LANES), f32)
    BX10 = jnp.zeros((8, _LANES), f32)
    BY20 = jnp.zeros((8, _LANES), f32)
    BX20 = jnp.zeros((8, _LANES), f32)
    for j in range(_NB):
        vj = s0[j]
        bj = sbits[j]
        better = (vj > M0) | ((vj == M0) & (bj > SB0))
        M0 = jnp.where(better, vj, M0)
        SB0 = jnp.where(better, bj, SB0)
        PS0 = jnp.where(better, jnp.int32(j) * _BLK + pos, PS0)
        BY10 = jnp.where(better, y1[j], BY10)
        BX10 = jnp.where(better, x1[j], BX10)
        BY20 = jnp.where(better, y2[j], BY20)
        BX20 = jnp.where(better, x2[j], BX20)
    m0 = jnp.max(M0)

    zeros8 = jnp.zeros((8, _LANES), jnp.float32)
    _NACC = 8

    def _one(i, carry, src_ref, dst_ref):
        ox1, oy1, ox2, oy2, M, SB, PS, BY1, BX1, BY2, BX2, m = carry
        tiedp = M == m
        tcnt = jnp.sum(tiedp.astype(jnp.int32))
        pidx0 = jnp.min(jnp.where(tiedp, PS, jnp.int32(_NPAD)))
        tf = tiedp.astype(f32)
        sy1 = jnp.sum(BY1 * tf)
        sx1 = jnp.sum(BX1 * tf)
        sy2 = jnp.sum(BY2 * tf)
        sx2 = jnp.sum(BX2 * tf)

        def _exact(_):
            msb = jnp.max(jnp.where(tiedp, SB, jnp.int32(-1)))
            tied2 = tiedp & (SB == msb)
            p = jnp.min(jnp.where(tied2, PS, jnp.int32(_NPAD)))
            w = (PS == p) & tied2
            wf = w.astype(f32)
            return (p, jnp.sum(BY1 * wf), jnp.sum(BX1 * wf),
                    jnp.sum(BY2 * wf), jnp.sum(BX2 * wf))

        def _fast(_):
            return (pidx0, sy1, sx1, sy2, sx2)

        pidx, by1, bx1, by2, bx2 = jax.lax.cond(tcnt == 1, _fast, _exact, 0)
        ok = m > f32(_NEG_INF)
        here = pos == i
        zf = f32(0.0)
        ox1 = jnp.where(here, jnp.where(ok, bx1, zf), ox1)
        oy1 = jnp.where(here, jnp.where(ok, by1, zf), oy1)
        ox2 = jnp.where(here, jnp.where(ok, bx2, zf), ox2)
        oy2 = jnp.where(here, jnp.where(ok, by2, zf), oy2)
        area_a = (by2 - by1) * (bx2 - bx1)
        Ms = [jnp.full((8, _LANES), f32(_NEG_INF)) for _ in range(_NACC)]
        SBs = [jnp.full((8, _LANES), jnp.int32(-1)) for _ in range(_NACC)]
        PSs = [jnp.zeros((8, _LANES), jnp.int32) for _ in range(_NACC)]
        Y1s = [jnp.zeros((8, _LANES), f32) for _ in range(_NACC)]
        X1s = [jnp.zeros((8, _LANES), f32) for _ in range(_NACC)]
        Y2s = [jnp.zeros((8, _LANES), f32) for _ in range(_NACC)]
        X2s = [jnp.zeros((8, _LANES), f32) for _ in range(_NACC)]
        for j in range(_NB):
            a = j % _NACC
            sj = src_ref[j]
            cy1 = y1_ref[j]
            cx1 = x1_ref[j]
            cy2 = y2_ref[j]
            cx2 = x2_ref[j]
            yy1 = jnp.maximum(by1, cy1)
            xx1 = jnp.maximum(bx1, cx1)
            yy2 = jnp.minimum(by2, cy2)
            xx2 = jnp.minimum(bx2, cx2)
            inter = jnp.maximum(yy2 - yy1, f32(0.0)) * \
                jnp.maximum(xx2 - xx1, f32(0.0))
            ious = inter / (area_a + ar_ref[j] - inter + f32(1e-8))
            decay = jnp.exp(-(ious * ious) / f32(_SIGMA))
            v2 = sj * decay
            v2 = jnp.where(pos == pidx - j * _BLK, f32(_NEG_INF), v2)
            dst_ref[j] = v2
            bj = sb_ref[j]
            better = (v2 > Ms[a]) | ((v2 == Ms[a]) & (bj > SBs[a]))
            Ms[a] = jnp.where(better, v2, Ms[a])
            SBs[a] = jnp.where(better, bj, SBs[a])
            PSs[a] = jnp.where(better, jnp.int32(j) * _BLK + pos, PSs[a])
            Y1s[a] = jnp.where(better, cy1, Y1s[a])
            X1s[a] = jnp.where(better, cx1, X1s[a])
            Y2s[a] = jnp.where(better, cy2, Y2s[a])
            X2s[a] = jnp.where(better, cx2, X2s[a])
        Mn, SBn, PSn = Ms[0], SBs[0], PSs[0]
        Y1n, X1n, Y2n, X2n = Y1s[0], X1s[0], Y2s[0], X2s[0]
        for a in range(1, _NACC):
            better = (Ms[a] > Mn) | ((Ms[a] == Mn) & (
                (SBs[a] > SBn) | ((SBs[a] == SBn) & (PSs[a] < PSn))))
            Mn = jnp.where(better, Ms[a], Mn)
            SBn = jnp.where(better, SBs[a], SBn)
            PSn = jnp.where(better, PSs[a], PSn)
            Y1n = jnp.where(better, Y1s[a], Y1n)
            X1n = jnp.where(better, X1s[a], X1n)
            Y2n = jnp.where(better, Y2s[a], Y2n)
            X2n = jnp.where(better, X2s[a], X2n)
        mtree = list(Ms)
        while len(mtree) > 1:
            mtree = [jnp.maximum(mtree[t], mtree[t + 1])
                     for t in range(0, len(mtree) - 1, 2)] + \
                (mtree[-1:] if len(mtree) % 2 else [])
        mn = jnp.max(mtree[0])
        return ox1, oy1, ox2, oy2, Mn, SBn, PSn, Y1n, X1n, Y2n, X2n, mn

    def _iter2(k, carry):
        carry = _one(2 * k, carry, s_ref, s2_ref)
        return _one(2 * k + 1, carry, s2_ref, s_ref)

    ox1, oy1, ox2, oy2 = jax.lax.fori_loop(
        0, _OUT // 2, _iter2,
        (zeros8, zeros8, zeros8, zeros8, M0, SB0, PS0,
         BY10, BX10, BY20, BX20, m0))[:4]
    out_ref[0] = ox1
    out_ref[1] = oy1
    out_ref[2] = ox2
    out_ref[3] = oy2


@functools.partial(jax.jit, static_argnames=("interpret",))
def _run(rpn_score, rpn_regress, anchors, interpret=False):
    score = rpn_score.reshape(_N)
    pad = _NPAD - _N
    score_p = jnp.pad(score, (0, pad)).reshape(_NB, 8, _LANES)
    anch_p = jnp.pad(anchors, ((0, pad), (0, 0))).T.reshape(4, _NB, 8, _LANES)
    delt_p = jnp.pad(rpn_regress.reshape(_N, 4),
                     ((0, pad), (0, 0))).T.reshape(4, _NB, 8, _LANES)
    out = pl.pallas_call(
        _nms_body,
        out_shape=jax.ShapeDtypeStruct((4, 8, _LANES), jnp.float32),
        scratch_shapes=[
            pltpu.VMEM((_NB, 8, _LANES), jnp.float32),
            pltpu.VMEM((_NB, 8, _LANES), jnp.float32),
            pltpu.VMEM((_NB, 8, _LANES), jnp.int32),
            pltpu.VMEM((_NB, 8, _LANES), jnp.float32),
            pltpu.VMEM((_NB, 8, _LANES), jnp.float32),
            pltpu.VMEM((_NB, 8, _LANES), jnp.float32),
            pltpu.VMEM((_NB, 8, _LANES), jnp.float32),
            pltpu.VMEM((_NB, 8, _LANES), jnp.float32),
        ],
        interpret=interpret,
    )(score_p, anch_p, delt_p)
    out = out.reshape(4, 8 * _LANES)[:, :_OUT]
    return jax.lax.stop_gradient(out.T.reshape(1, _OUT, 4))


def kernel(rpn_score, rpn_regress, anchors):
    return _run(rpn_score, rpn_regress, anchors)


# speedup vs baseline: 1.8545x; 1.0089x over previous
import functools

import jax
import jax.numpy as jnp
import numpy as np
from jax.experimental import pallas as pl
from jax.experimental.pallas import tpu as pltpu

_N = 20000
_NB = 20
_BLK = 1024
_LANES = 128
_NPAD = _NB * _BLK
_K = 5000
_OUT = 1000
_NEG_INF = float("-inf")
_MAX_RATIO = float(np.abs(np.log(np.float32(16.0 / 1000.0))))
_SIGMA = 0.5


def _nms_body(score_ref, anch_ref, delt_ref, out_ref,
              s_ref, s2_ref, sb_ref, y1_ref, x1_ref, y2_ref, x2_ref, ar_ref):
    f32 = jnp.float32
    sc = score_ref[:]

    a0 = anch_ref[0]
    a1 = anch_ref[1]
    a2 = anch_ref[2]
    a3 = anch_ref[3]
    dx = delt_ref[0] * f32(0.1)
    dy = delt_ref[1] * f32(0.1)
    dw = jnp.clip(delt_ref[2] * f32(0.2), -_MAX_RATIO, _MAX_RATIO)
    dh = jnp.clip(delt_ref[3] * f32(0.2), -_MAX_RATIO, _MAX_RATIO)
    widths = a2 - a0
    heights = a3 - a1
    ctr_x = a0 + f32(0.5) * widths
    ctr_y = a1 + f32(0.5) * heights
    pred_ctr_x = dx * widths + ctr_x
    pred_ctr_y = dy * heights + ctr_y
    pred_w = widths * jnp.exp(dw)
    pred_h = heights * jnp.exp(dh)
    x1 = jnp.clip(pred_ctr_x - f32(0.5) * pred_w, 0.0, 1.0)
    y1 = jnp.clip(pred_ctr_y - f32(0.5) * pred_h, 0.0, 1.0)
    x2 = jnp.clip(pred_ctr_x + f32(0.5) * pred_w, 0.0, 1.0)
    y2 = jnp.clip(pred_ctr_y + f32(0.5) * pred_h, 0.0, 1.0)
    areas = (y2 - y1) * (x2 - x1)
    y1_ref[:] = y1
    x1_ref[:] = x1
    y2_ref[:] = y2
    x2_ref[:] = x2
    ar_ref[:] = areas

    jb = jax.lax.broadcasted_iota(jnp.int32, (_NB, 8, _LANES), 0)
    sub = jax.lax.broadcasted_iota(jnp.int32, (_NB, 8, _LANES), 1)
    lane = jax.lax.broadcasted_iota(jnp.int32, (_NB, 8, _LANES), 2)
    idx3 = jb * _BLK + sub * _LANES + lane
    valid = idx3 < _N
    sbits = jnp.where(valid, jax.lax.bitcast_convert_type(sc, jnp.int32),
                      jnp.int32(-1))
    sb_ref[:] = sbits

    def _bs_val(_, c):
        lo, hi = c
        mid = (lo + hi) // 2
        cnt = jnp.sum((sbits >= mid).astype(jnp.int32))
        big = cnt >= _K
        return jnp.where(big, mid, lo), jnp.where(big, hi, mid)

    v, _ = jax.lax.fori_loop(
        0, 31, _bs_val, (jnp.int32(0), jnp.int32(0x3F800000)))
    cnt_gt = jnp.sum((sbits > v).astype(jnp.int32))
    need = _K - cnt_gt
    eq = sbits == v

    def _bs_idx(_, c):
        lo, hi = c
        mid = (lo + hi) // 2
        g = jnp.sum((eq & (idx3 < mid)).astype(jnp.int32))
        enough = g >= need
        return jnp.where(enough, lo, mid), jnp.where(enough, mid, hi)

    _, cut = jax.lax.fori_loop(
        0, 16, _bs_idx, (jnp.int32(0), jnp.int32(_NPAD)))
    sel = (sbits > v) | (eq & (idx3 < cut))
    s0 = jnp.where(sel, sc, f32(_NEG_INF))
    s_ref[:] = s0

    pos = jax.lax.broadcasted_iota(jnp.int32, (8, _LANES), 0) * _LANES + \
        jax.lax.broadcasted_iota(jnp.int32, (8, _LANES), 1)
    M0 = jnp.full((8, _LANES), f32(_NEG_INF))
    SB0 = jnp.full((8, _LANES), jnp.int32(-1))
    PS0 = jnp.zeros((8, _LANES), jnp.int32)
    BY10 = jnp.zeros((8, _LANES), f32)
    BX10 = jnp.zeros((8, _LANES), f32)
    BY20 = jnp.zeros((8, _LANES), f32)
    BX20 = jnp.zeros((8, _LANES), f32)
    for j in range(_NB):
        vj = s0[j]
        bj = sbits[j]
        better = (vj > M0) | ((vj == M0) & (bj > SB0))
        M0 = jnp.where(better, vj, M0)
        SB0 = jnp.where(better, bj, SB0)
        PS0 = jnp.where(better, jnp.int32(j) * _BLK + pos, PS0)
        BY10 = jnp.where(better, y1[j], BY10)
        BX10 = jnp.where(better, x1[j], BX10)
        BY20 = jnp.where(better, y2[j], BY20)
        BX20 = jnp.where(better, x2[j], BX20)
    m0 = jnp.max(M0)

    zeros8 = jnp.zeros((8, _LANES), jnp.float32)
    _NACC = 4

    def _one(i, carry, src_ref, dst_ref):
        ox1, oy1, ox2, oy2, M, SB, PS, BY1, BX1, BY2, BX2, m = carry
        tiedp = M == m
        tcnt = jnp.sum(tiedp.astype(jnp.int32))
        pidx0 = jnp.min(jnp.where(tiedp, PS, jnp.int32(_NPAD)))
        tf = tiedp.astype(f32)
        sy1 = jnp.sum(BY1 * tf)
        sx1 = jnp.sum(BX1 * tf)
        sy2 = jnp.sum(BY2 * tf)
        sx2 = jnp.sum(BX2 * tf)

        def _exact(_):
            msb = jnp.max(jnp.where(tiedp, SB, jnp.int32(-1)))
            tied2 = tiedp & (SB == msb)
            p = jnp.min(jnp.where(tied2, PS, jnp.int32(_NPAD)))
            w = (PS == p) & tied2
            wf = w.astype(f32)
            return (p, jnp.sum(BY1 * wf), jnp.sum(BX1 * wf),
                    jnp.sum(BY2 * wf), jnp.sum(BX2 * wf))

        def _fast(_):
            return (pidx0, sy1, sx1, sy2, sx2)

        pidx, by1, bx1, by2, bx2 = jax.lax.cond(tcnt == 1, _fast, _exact, 0)
        ok = m > f32(_NEG_INF)
        here = pos == i
        zf = f32(0.0)
        ox1 = jnp.where(here, jnp.where(ok, bx1, zf), ox1)
        oy1 = jnp.where(here, jnp.where(ok, by1, zf), oy1)
        ox2 = jnp.where(here, jnp.where(ok, bx2, zf), ox2)
        oy2 = jnp.where(here, jnp.where(ok, by2, zf), oy2)
        area_a = (by2 - by1) * (bx2 - bx1)
        Ms = [jnp.full((8, _LANES), f32(_NEG_INF)) for _ in range(_NACC)]
        SBs = [jnp.full((8, _LANES), jnp.int32(-1)) for _ in range(_NACC)]
        PSs = [jnp.zeros((8, _LANES), jnp.int32) for _ in range(_NACC)]
        Y1s = [jnp.zeros((8, _LANES), f32) for _ in range(_NACC)]
        X1s = [jnp.zeros((8, _LANES), f32) for _ in range(_NACC)]
        Y2s = [jnp.zeros((8, _LANES), f32) for _ in range(_NACC)]
        X2s = [jnp.zeros((8, _LANES), f32) for _ in range(_NACC)]
        for j in range(_NB):
            a = j % _NACC
            sj = src_ref[j]
            cy1 = y1_ref[j]
            cx1 = x1_ref[j]
            cy2 = y2_ref[j]
            cx2 = x2_ref[j]
            yy1 = jnp.maximum(by1, cy1)
            xx1 = jnp.maximum(bx1, cx1)
            yy2 = jnp.minimum(by2, cy2)
            xx2 = jnp.minimum(bx2, cx2)
            inter = jnp.maximum(yy2 - yy1, f32(0.0)) * \
                jnp.maximum(xx2 - xx1, f32(0.0))
            ious = inter / (area_a + ar_ref[j] - inter + f32(1e-8))
            decay = jnp.exp(-(ious * ious) / f32(_SIGMA))
            v2 = sj * decay
            v2 = jnp.where(pos == pidx - j * _BLK, f32(_NEG_INF), v2)
            dst_ref[j] = v2
            bj = sb_ref[j]
            better = (v2 > Ms[a]) | ((v2 == Ms[a]) & (bj > SBs[a]))
            Ms[a] = jnp.where(better, v2, Ms[a])
            SBs[a] = jnp.where(better, bj, SBs[a])
            PSs[a] = jnp.where(better, jnp.int32(j) * _BLK + pos, PSs[a])
            Y1s[a] = jnp.where(better, cy1, Y1s[a])
            X1s[a] = jnp.where(better, cx1, X1s[a])
            Y2s[a] = jnp.where(better, cy2, Y2s[a])
            X2s[a] = jnp.where(better, cx2, X2s[a])
        Mn, SBn, PSn = Ms[0], SBs[0], PSs[0]
        Y1n, X1n, Y2n, X2n = Y1s[0], X1s[0], Y2s[0], X2s[0]
        for a in range(1, _NACC):
            better = (Ms[a] > Mn) | ((Ms[a] == Mn) & (
                (SBs[a] > SBn) | ((SBs[a] == SBn) & (PSs[a] < PSn))))
            Mn = jnp.where(better, Ms[a], Mn)
            SBn = jnp.where(better, SBs[a], SBn)
            PSn = jnp.where(better, PSs[a], PSn)
            Y1n = jnp.where(better, Y1s[a], Y1n)
            X1n = jnp.where(better, X1s[a], X1n)
            Y2n = jnp.where(better, Y2s[a], Y2n)
            X2n = jnp.where(better, X2s[a], X2n)
        mtree = list(Ms)
        while len(mtree) > 1:
            mtree = [jnp.maximum(mtree[t], mtree[t + 1])
                     for t in range(0, len(mtree) - 1, 2)] + \
                (mtree[-1:] if len(mtree) % 2 else [])
        mn = jnp.max(mtree[0])
        return ox1, oy1, ox2, oy2, Mn, SBn, PSn, Y1n, X1n, Y2n, X2n, mn

    def _iter2(k, carry):
        carry = _one(2 * k, carry, s_ref, s2_ref)
        return _one(2 * k + 1, carry, s2_ref, s_ref)

    ox1, oy1, ox2, oy2 = jax.lax.fori_loop(
        0, _OUT // 2, _iter2,
        (zeros8, zeros8, zeros8, zeros8, M0, SB0, PS0,
         BY10, BX10, BY20, BX20, m0))[:4]
    out_ref[0] = ox1
    out_ref[1] = oy1
    out_ref[2] = ox2
    out_ref[3] = oy2


@functools.partial(jax.jit, static_argnames=("interpret",))
def _run(rpn_score, rpn_regress, anchors, interpret=False):
    score = rpn_score.reshape(_N)
    pad = _NPAD - _N
    score_p = jnp.pad(score, (0, pad)).reshape(_NB, 8, _LANES)
    anch_p = jnp.pad(anchors, ((0, pad), (0, 0))).T.reshape(4, _NB, 8, _LANES)
    delt_p = jnp.pad(rpn_regress.reshape(_N, 4),
                     ((0, pad), (0, 0))).T.reshape(4, _NB, 8, _LANES)
    out = pl.pallas_call(
        _nms_body,
        out_shape=jax.ShapeDtypeStruct((4, 8, _LANES), jnp.float32),
        scratch_shapes=[
            pltpu.VMEM((_NB, 8, _LANES), jnp.float32),
            pltpu.VMEM((_NB, 8, _LANES), jnp.float32),
            pltpu.VMEM((_NB, 8, _LANES), jnp.int32),
            pltpu.VMEM((_NB, 8, _LANES), jnp.float32),
            pltpu.VMEM((_NB, 8, _LANES), jnp.float32),
            pltpu.VMEM((_NB, 8, _LANES), jnp.float32),
            pltpu.VMEM((_NB, 8, _LANES), jnp.float32),
            pltpu.VMEM((_NB, 8, _LANES), jnp.float32),
        ],
        interpret=interpret,
    )(score_p, anch_p, delt_p)
    out = out.reshape(4, 8 * _LANES)[:, :_OUT]
    return jax.lax.stop_gradient(out.T.reshape(1, _OUT, 4))


def kernel(rpn_score, rpn_regress, anchors):
    return _run(rpn_score, rpn_regress, anchors)


# speedup vs baseline: 1.8634x; 1.0048x over previous
import jax
import jax.numpy as jnp
import numpy as np
from jax.experimental import pallas as pl
from jax.experimental.pallas import tpu as pltpu

_N = 20000
_NB = 20
_BLK = 1024
_LANES = 128
_NPAD = _NB * _BLK
_K = 5000
_OUT = 1000
_NEG_INF = float("-inf")
_MAX_RATIO = float(np.abs(np.log(np.float32(16.0 / 1000.0))))
_SIGMA = 0.5


def _nms_body(score_ref, anch_ref, delt_ref, out_ref,
              s_ref, s2_ref, sb_ref, y1_ref, x1_ref, y2_ref, x2_ref, ar_ref):
    f32 = jnp.float32
    sc = score_ref[:]

    a0 = anch_ref[0]
    a1 = anch_ref[1]
    a2 = anch_ref[2]
    a3 = anch_ref[3]
    dx = delt_ref[0] * f32(0.1)
    dy = delt_ref[1] * f32(0.1)
    dw = jnp.clip(delt_ref[2] * f32(0.2), -_MAX_RATIO, _MAX_RATIO)
    dh = jnp.clip(delt_ref[3] * f32(0.2), -_MAX_RATIO, _MAX_RATIO)
    widths = a2 - a0
    heights = a3 - a1
    ctr_x = a0 + f32(0.5) * widths
    ctr_y = a1 + f32(0.5) * heights
    pred_ctr_x = dx * widths + ctr_x
    pred_ctr_y = dy * heights + ctr_y
    pred_w = widths * jnp.exp(dw)
    pred_h = heights * jnp.exp(dh)
    x1 = jnp.clip(pred_ctr_x - f32(0.5) * pred_w, 0.0, 1.0)
    y1 = jnp.clip(pred_ctr_y - f32(0.5) * pred_h, 0.0, 1.0)
    x2 = jnp.clip(pred_ctr_x + f32(0.5) * pred_w, 0.0, 1.0)
    y2 = jnp.clip(pred_ctr_y + f32(0.5) * pred_h, 0.0, 1.0)
    areas = (y2 - y1) * (x2 - x1)
    y1_ref[:] = y1
    x1_ref[:] = x1
    y2_ref[:] = y2
    x2_ref[:] = x2
    ar_ref[:] = areas

    jb = jax.lax.broadcasted_iota(jnp.int32, (_NB, 8, _LANES), 0)
    sub = jax.lax.broadcasted_iota(jnp.int32, (_NB, 8, _LANES), 1)
    lane = jax.lax.broadcasted_iota(jnp.int32, (_NB, 8, _LANES), 2)
    idx3 = jb * _BLK + sub * _LANES + lane
    valid = idx3 < _N
    sbits = jnp.where(valid, jax.lax.bitcast_convert_type(sc, jnp.int32),
                      jnp.int32(-1))
    sb_ref[:] = sbits

    def _bs_val(_, c):
        lo, hi = c
        mid = (lo + hi) // 2
        cnt = jnp.sum((sbits >= mid).astype(jnp.int32))
        big = cnt >= _K
        return jnp.where(big, mid, lo), jnp.where(big, hi, mid)

    v, _ = jax.lax.fori_loop(
        0, 31, _bs_val, (jnp.int32(0), jnp.int32(0x3F800000)))
    cnt_gt = jnp.sum((sbits > v).astype(jnp.int32))
    need = _K - cnt_gt
    eq = sbits == v

    def _bs_idx(_, c):
        lo, hi = c
        mid = (lo + hi) // 2
        g = jnp.sum((eq & (idx3 < mid)).astype(jnp.int32))
        enough = g >= need
        return jnp.where(enough, lo, mid), jnp.where(enough, mid, hi)

    _, cut = jax.lax.fori_loop(
        0, 16, _bs_idx, (jnp.int32(0), jnp.int32(_NPAD)))
    sel = (sbits > v) | (eq & (idx3 < cut))
    s0 = jnp.where(sel, sc, f32(_NEG_INF))
    s_ref[:] = s0

    pos = jax.lax.broadcasted_iota(jnp.int32, (8, _LANES), 0) * _LANES + \
        jax.lax.broadcasted_iota(jnp.int32, (8, _LANES), 1)
    M0 = jnp.full((8, _LANES), f32(_NEG_INF))
    SB0 = jnp.full((8, _LANES), jnp.int32(-1))
    PS0 = jnp.zeros((8, _LANES), jnp.int32)
    BY10 = jnp.zeros((8, _LANES), f32)
    BX10 = jnp.zeros((8, _LANES), f32)
    BY20 = jnp.zeros((8, _LANES), f32)
    BX20 = jnp.zeros((8, _LANES), f32)
    for j in range(_NB):
        vj = s0[j]
        bj = sbits[j]
        better = (vj > M0) | ((vj == M0) & (bj > SB0))
        M0 = jnp.where(better, vj, M0)
        SB0 = jnp.where(better, bj, SB0)
        PS0 = jnp.where(better, jnp.int32(j) * _BLK + pos, PS0)
        BY10 = jnp.where(better, y1[j], BY10)
        BX10 = jnp.where(better, x1[j], BX10)
        BY20 = jnp.where(better, y2[j], BY20)
        BX20 = jnp.where(better, x2[j], BX20)
    m0 = jnp.max(M0)

    zeros8 = jnp.zeros((8, _LANES), jnp.float32)
    _NACC = 4

    def _one(i, carry, src_ref, dst_ref):
        ox1, oy1, ox2, oy2, M, SB, PS, BY1, BX1, BY2, BX2, m = carry
        tiedp = M == m
        tcnt = jnp.sum(tiedp.astype(jnp.int32))
        pidx0 = jnp.min(jnp.where(tiedp, PS, jnp.int32(_NPAD)))
        tf = tiedp.astype(f32)
        sy1 = jnp.sum(BY1 * tf)
        sx1 = jnp.sum(BX1 * tf)
        sy2 = jnp.sum(BY2 * tf)
        sx2 = jnp.sum(BX2 * tf)

        def _exact(_):
            msb = jnp.max(jnp.where(tiedp, SB, jnp.int32(-1)))
            tied2 = tiedp & (SB == msb)
            p = jnp.min(jnp.where(tied2, PS, jnp.int32(_NPAD)))
            w = (PS == p) & tied2
            wf = w.astype(f32)
            return (p, jnp.sum(BY1 * wf), jnp.sum(BX1 * wf),
                    jnp.sum(BY2 * wf), jnp.sum(BX2 * wf))

        def _fast(_):
            return (pidx0, sy1, sx1, sy2, sx2)

        pidx, by1, bx1, by2, bx2 = jax.lax.cond(tcnt == 1, _fast, _exact, 0)
        ok = m > f32(_NEG_INF)
        here = pos == i
        zf = f32(0.0)
        ox1 = jnp.where(here, jnp.where(ok, bx1, zf), ox1)
        oy1 = jnp.where(here, jnp.where(ok, by1, zf), oy1)
        ox2 = jnp.where(here, jnp.where(ok, bx2, zf), ox2)
        oy2 = jnp.where(here, jnp.where(ok, by2, zf), oy2)
        area_a = (by2 - by1) * (bx2 - bx1)
        Ms = [jnp.full((8, _LANES), f32(_NEG_INF)) for _ in range(_NACC)]
        SBs = [jnp.full((8, _LANES), jnp.int32(-1)) for _ in range(_NACC)]
        PSs = [jnp.zeros((8, _LANES), jnp.int32) for _ in range(_NACC)]
        Y1s = [jnp.zeros((8, _LANES), f32) for _ in range(_NACC)]
        X1s = [jnp.zeros((8, _LANES), f32) for _ in range(_NACC)]
        Y2s = [jnp.zeros((8, _LANES), f32) for _ in range(_NACC)]
        X2s = [jnp.zeros((8, _LANES), f32) for _ in range(_NACC)]
        for j in range(_NB):
            a = j % _NACC
            sj = src_ref[j]
            cy1 = y1_ref[j]
            cx1 = x1_ref[j]
            cy2 = y2_ref[j]
            cx2 = x2_ref[j]
            yy1 = jnp.maximum(by1, cy1)
            xx1 = jnp.maximum(bx1, cx1)
            yy2 = jnp.minimum(by2, cy2)
            xx2 = jnp.minimum(bx2, cx2)
            inter = jnp.maximum(yy2 - yy1, f32(0.0)) * \
                jnp.maximum(xx2 - xx1, f32(0.0))
            ious = inter / (area_a + ar_ref[j] - inter + f32(1e-8))
            decay = jnp.exp(-(ious * ious) / f32(_SIGMA))
            v2 = sj * decay
            v2 = jnp.where(pos == pidx - j * _BLK, f32(_NEG_INF), v2)
            dst_ref[j] = v2
            bj = sb_ref[j]
            better = (v2 > Ms[a]) | ((v2 == Ms[a]) & (bj > SBs[a]))
            Ms[a] = jnp.where(better, v2, Ms[a])
            SBs[a] = jnp.where(better, bj, SBs[a])
            PSs[a] = jnp.where(better, jnp.int32(j) * _BLK + pos, PSs[a])
            Y1s[a] = jnp.where(better, cy1, Y1s[a])
            X1s[a] = jnp.where(better, cx1, X1s[a])
            Y2s[a] = jnp.where(better, cy2, Y2s[a])
            X2s[a] = jnp.where(better, cx2, X2s[a])
        Mn, SBn, PSn = Ms[0], SBs[0], PSs[0]
        Y1n, X1n, Y2n, X2n = Y1s[0], X1s[0], Y2s[0], X2s[0]
        for a in range(1, _NACC):
            better = (Ms[a] > Mn) | ((Ms[a] == Mn) & (
                (SBs[a] > SBn) | ((SBs[a] == SBn) & (PSs[a] < PSn))))
            Mn = jnp.where(better, Ms[a], Mn)
            SBn = jnp.where(better, SBs[a], SBn)
            PSn = jnp.where(better, PSs[a], PSn)
            Y1n = jnp.where(better, Y1s[a], Y1n)
            X1n = jnp.where(better, X1s[a], X1n)
            Y2n = jnp.where(better, Y2s[a], Y2n)
            X2n = jnp.where(better, X2s[a], X2n)
        mtree = list(Ms)
        while len(mtree) > 1:
            mtree = [jnp.maximum(mtree[t], mtree[t + 1])
                     for t in range(0, len(mtree) - 1, 2)] + \
                (mtree[-1:] if len(mtree) % 2 else [])
        mn = jnp.max(mtree[0])
        return ox1, oy1, ox2, oy2, Mn, SBn, PSn, Y1n, X1n, Y2n, X2n, mn

    def _iter2(k, carry):
        carry = _one(2 * k, carry, s_ref, s2_ref)
        return _one(2 * k + 1, carry, s2_ref, s_ref)

    ox1, oy1, ox2, oy2 = jax.lax.fori_loop(
        0, _OUT // 2, _iter2,
        (zeros8, zeros8, zeros8, zeros8, M0, SB0, PS0,
         BY10, BX10, BY20, BX20, m0))[:4]
    out_ref[0] = ox1
    out_ref[1] = oy1
    out_ref[2] = ox2
    out_ref[3] = oy2


@jax.jit
def _run(rpn_score, rpn_regress, anchors):
    score = rpn_score.reshape(_N)
    pad = _NPAD - _N
    score_p = jnp.pad(score, (0, pad)).reshape(_NB, 8, _LANES)
    anch_p = jnp.pad(anchors, ((0, pad), (0, 0))).T.reshape(4, _NB, 8, _LANES)
    delt_p = jnp.pad(rpn_regress.reshape(_N, 4),
                     ((0, pad), (0, 0))).T.reshape(4, _NB, 8, _LANES)
    out = pl.pallas_call(
        _nms_body,
        out_shape=jax.ShapeDtypeStruct((4, 8, _LANES), jnp.float32),
        scratch_shapes=[
            pltpu.VMEM((_NB, 8, _LANES), jnp.float32),
            pltpu.VMEM((_NB, 8, _LANES), jnp.float32),
            pltpu.VMEM((_NB, 8, _LANES), jnp.int32),
            pltpu.VMEM((_NB, 8, _LANES), jnp.float32),
            pltpu.VMEM((_NB, 8, _LANES), jnp.float32),
            pltpu.VMEM((_NB, 8, _LANES), jnp.float32),
            pltpu.VMEM((_NB, 8, _LANES), jnp.float32),
            pltpu.VMEM((_NB, 8, _LANES), jnp.float32),
        ],
    )(score_p, anch_p, delt_p)
    out = out.reshape(4, 8 * _LANES)[:, :_OUT]
    return jax.lax.stop_gradient(out.T.reshape(1, _OUT, 4))


def kernel(rpn_score, rpn_regress, anchors):
    return _run(rpn_score, rpn_regress, anchors)
